# baseline (speedup 1.0000x reference)
import jax
import jax.numpy as jnp
from jax.experimental import pallas as pl
from jax.experimental.pallas import tpu as pltpu

BUCKET_SIZE = 32
DIM = 128


def _bucket_mean(x):
    n, d = x.shape
    buckets = n // BUCKET_SIZE
    return jnp.sum(x.reshape(buckets, BUCKET_SIZE, d), axis=1) * (
        1.0 / BUCKET_SIZE)


def _body(q1_ref, q2_ref, k1_ref, k2_ref, o_ref):
    sq = jnp.concatenate(
        [_bucket_mean(q1_ref[0]), _bucket_mean(q2_ref[0])], axis=0)
    sk = jnp.concatenate(
        [_bucket_mean(k1_ref[0]), _bucket_mean(k2_ref[0])], axis=0)
    r = jax.lax.dot_general(
        sq, sk, (((1,), (1,)), ((), ())),
        preferred_element_type=jnp.float32) * (DIM ** -0.5)
    m = jnp.max(r, axis=-1, keepdims=True)
    e = jnp.exp(r - m)
    o_ref[0] = e / jnp.sum(e, axis=-1, keepdims=True)


def kernel(q, k):
    bh, n, d = q.shape
    buckets = n // BUCKET_SIZE
    h = n // 2
    lo = pl.BlockSpec((1, h, d), lambda i: (i, 0, 0))
    hi = pl.BlockSpec((1, h, d), lambda i: (i, 1, 0))
    return pl.pallas_call(
        _body,
        grid=(bh,),
        in_specs=[lo, hi, lo, hi],
        out_specs=pl.BlockSpec((1, buckets, buckets), lambda i: (i, 0, 0)),
        out_shape=jax.ShapeDtypeStruct((bh, buckets, buckets), jnp.float32),
    )(q, q, k, k)
